# baseline (speedup 1.0000x reference)
import functools

import numpy as np
import jax
import jax.numpy as jnp
from jax import lax
from jax.experimental import pallas as pl
from jax.experimental.pallas import tpu as pltpu
from jax.experimental.pallas import tpu_sc as plsc

_T = 0.1
_K = 16
_B = 128
_N = 32768
_ROWS = 8
_LANES = 16
_NVREG = _N // _LANES
_SC_WORKERS = 32
_ROWS_PER_W = _B // _SC_WORKERS

_EPS = float(np.finfo(np.float32).eps)
_NEG = float(np.float32(-np.inf))


def _threefry2x32(ks0, ks1, x0, x1):
    rot = (np.array([13, 15, 26, 6], np.uint32), np.array([17, 29, 16, 24], np.uint32))
    ks2 = np.uint32(ks0 ^ ks1 ^ np.uint32(0x1BD11BDA))
    ks = (np.uint32(ks0), np.uint32(ks1), ks2)

    def rotl(v, d):
        return (v << np.uint32(d)) | (v >> np.uint32(32 - d))

    def rnds(x0, x1, ds):
        for d in ds:
            x0 = x0 + x1
            x1 = rotl(x1, d)
            x1 = x0 ^ x1
        return x0, x1

    x0 = x0 + ks[0]
    x1 = x1 + ks[1]
    x0, x1 = rnds(x0, x1, rot[0])
    x0 = x0 + ks[1]
    x1 = x1 + ks[2] + np.uint32(1)
    x0, x1 = rnds(x0, x1, rot[1])
    x0 = x0 + ks[2]
    x1 = x1 + ks[0] + np.uint32(2)
    x0, x1 = rnds(x0, x1, rot[0])
    x0 = x0 + ks[0]
    x1 = x1 + ks[1] + np.uint32(3)
    x0, x1 = rnds(x0, x1, rot[1])
    x0 = x0 + ks[1]
    x1 = x1 + ks[2] + np.uint32(4)
    x0, x1 = rnds(x0, x1, rot[0])
    x0 = x0 + ks[2]
    x1 = x1 + ks[0] + np.uint32(5)
    return x0, x1


def _gumbel_noise():
    size = _B * _N
    with np.errstate(over="ignore"):
        x0, x1 = _threefry2x32(
            np.uint32(0), np.uint32(42),
            np.zeros(size, np.uint32), np.arange(size, dtype=np.uint32))
    bits = x0 ^ x1
    u = ((bits >> np.uint32(9)) | np.uint32(0x3F800000)).view(np.float32) - np.float32(1.0)
    u = np.maximum(np.float32(0.0), u)
    u = np.clip(u, _EPS, 1.0 - _EPS).astype(np.float32)
    return (-np.log(-np.log(u))).reshape(_B, _N)


_NOISE_CONST = _gumbel_noise()


_GV = 8
_NG = _NVREG // _GV


def _sc_topk_body(x_hbm, d_hbm, row_v, out_v, sum_v):
    wid = lax.axis_index("s") * 2 + lax.axis_index("c")
    idx15 = jnp.full((_LANES,), _LANES - 1, jnp.int32)

    for rr in range(_ROWS_PER_W):
        r = wid * _ROWS_PER_W + rr
        pltpu.sync_copy(x_hbm.at[r], row_v)

        def p1(g, _):
            base = g * (_GV * _LANES)
            gm = row_v[pl.ds(base, _LANES)]
            for j in range(1, _GV):
                gm = jnp.maximum(gm, row_v[pl.ds(base + j * _LANES, _LANES)])
            sum_v[pl.ds(g * _LANES, _LANES)] = gm
            return 0

        lax.fori_loop(0, _NG, p1, 0)

        def p2(g, carry):
            cand, cmin = carry
            v = sum_v[pl.ds(g * _LANES, _LANES)]

            def merge(args):
                cand, _ = args
                vs = plsc.sort_key_val(v, v)[0]
                m = jnp.maximum(cand, vs)
                cand2 = plsc.sort_key_val(m, m, descending=True)[0]
                return cand2, jnp.take(cand2, idx15)

            def skip(args):
                return args

            return lax.cond(jnp.any(v >= cmin), merge, skip, (cand, cmin))

        cand0 = jnp.full((_LANES,), _NEG, jnp.float32)
        _, c16 = lax.fori_loop(0, _NG, p2,
                               (cand0, jnp.full((_LANES,), _NEG, jnp.float32)))

        def p3(g, cand):
            gm = sum_v[pl.ds(g * _LANES, _LANES)]

            def hit(cand):
                base = g * (_GV * _LANES)
                for j in range(_GV):
                    v = row_v[pl.ds(base + j * _LANES, _LANES)]
                    vs = plsc.sort_key_val(v, v)[0]
                    m = jnp.maximum(cand, vs)
                    cand = plsc.sort_key_val(m, m, descending=True)[0]
                return cand

            def skip(cand):
                return cand

            return lax.cond(jnp.any(gm >= c16), hit, skip, cand)

        cande = lax.fori_loop(0, _NG, p3, cand0)
        thr = jnp.take(cande, idx15)

        def p4(g, _):
            base = g * (_GV * _LANES)
            for j in range(_GV):
                sl = pl.ds(base + j * _LANES, _LANES)
                out_v[sl] = jnp.where(row_v[sl] >= thr,
                                      jnp.float32(1.0), jnp.float32(0.0))
            return 0

        lax.fori_loop(0, _NG, p4, 0)
        pltpu.sync_copy(out_v, d_hbm.at[r])


def _sc_topk(x):
    mesh = plsc.VectorSubcoreMesh(core_axis_name="c", subcore_axis_name="s")
    fn = functools.partial(
        pl.kernel,
        mesh=mesh,
        out_type=jax.ShapeDtypeStruct((_B, _N), jnp.float32),
        scratch_types=[
            pltpu.VMEM((_N,), jnp.float32),
            pltpu.VMEM((_N,), jnp.float32),
            pltpu.VMEM((_NG * _LANES,), jnp.float32),
        ],
        compiler_params=pltpu.CompilerParams(needs_layout_passes=False),
    )(_sc_topk_body)
    return fn(x)


def _soft_sub_kernel(x_ref, z_ref, c_ref, w_ref):
    w0 = x_ref[...] + z_ref[...]
    w_ref[...] = w0
    c_ref[...] = jnp.zeros((_ROWS, _N), jnp.float32)
    mw0 = jnp.max(w0, axis=1, keepdims=True)

    def cbody(_, mw):
        mws = mw / _T
        w = w_ref[...]
        s = jnp.sum(jnp.exp(w / _T - mws), axis=1, keepdims=True)
        w2 = w_ref[...]
        oh = jnp.exp(w2 / _T - mws) / s
        c_ref[...] = c_ref[...] + oh
        wn = w2 + jnp.log(jnp.clip(1.0 - oh, _EPS, 1.0 - _EPS))
        w_ref[...] = wn
        return jnp.max(wn, axis=1, keepdims=True)

    jax.lax.fori_loop(0, _K, cbody, mw0)


def _tc_relax(x, z):
    grid = (_B // _ROWS,)
    return pl.pallas_call(
        _soft_sub_kernel,
        grid=grid,
        in_specs=[
            pl.BlockSpec((_ROWS, _N), lambda i: (i, 0)),
            pl.BlockSpec((_ROWS, _N), lambda i: (i, 0)),
        ],
        out_specs=pl.BlockSpec((_ROWS, _N), lambda i: (i, 0)),
        out_shape=jax.ShapeDtypeStruct((_B, _N), jnp.float32),
        scratch_shapes=[
            pltpu.VMEM((_ROWS, _N), jnp.float32),
        ],
    )(x, z)


def kernel(logits):
    x = logits.reshape(_B, _N)
    z = jnp.asarray(_NOISE_CONST)
    dsamples = _sc_topk(x)
    csamples = _tc_relax(x, z)
    return dsamples, csamples


# speedup vs baseline: 1.3559x; 1.3559x over previous
import functools

import numpy as np
import jax
import jax.numpy as jnp
from jax import lax
from jax.experimental import pallas as pl
from jax.experimental.pallas import tpu as pltpu
from jax.experimental.pallas import tpu_sc as plsc

_T = 0.1
_K = 16
_B = 128
_N = 32768
_ROWS = 32
_LANES = 16
_NVREG = _N // _LANES
_SC_WORKERS = 32
_ROWS_PER_W = _B // _SC_WORKERS

_EPS = float(np.finfo(np.float32).eps)
_NEG = float(np.float32(-np.inf))


def _threefry2x32(ks0, ks1, x0, x1):
    rot = (np.array([13, 15, 26, 6], np.uint32), np.array([17, 29, 16, 24], np.uint32))
    ks2 = np.uint32(ks0 ^ ks1 ^ np.uint32(0x1BD11BDA))
    ks = (np.uint32(ks0), np.uint32(ks1), ks2)

    def rotl(v, d):
        return (v << np.uint32(d)) | (v >> np.uint32(32 - d))

    def rnds(x0, x1, ds):
        for d in ds:
            x0 = x0 + x1
            x1 = rotl(x1, d)
            x1 = x0 ^ x1
        return x0, x1

    x0 = x0 + ks[0]
    x1 = x1 + ks[1]
    x0, x1 = rnds(x0, x1, rot[0])
    x0 = x0 + ks[1]
    x1 = x1 + ks[2] + np.uint32(1)
    x0, x1 = rnds(x0, x1, rot[1])
    x0 = x0 + ks[2]
    x1 = x1 + ks[0] + np.uint32(2)
    x0, x1 = rnds(x0, x1, rot[0])
    x0 = x0 + ks[0]
    x1 = x1 + ks[1] + np.uint32(3)
    x0, x1 = rnds(x0, x1, rot[1])
    x0 = x0 + ks[1]
    x1 = x1 + ks[2] + np.uint32(4)
    x0, x1 = rnds(x0, x1, rot[0])
    x0 = x0 + ks[2]
    x1 = x1 + ks[0] + np.uint32(5)
    return x0, x1


def _gumbel_noise():
    size = _B * _N
    with np.errstate(over="ignore"):
        x0, x1 = _threefry2x32(
            np.uint32(0), np.uint32(42),
            np.zeros(size, np.uint32), np.arange(size, dtype=np.uint32))
    bits = x0 ^ x1
    u = ((bits >> np.uint32(9)) | np.uint32(0x3F800000)).view(np.float32) - np.float32(1.0)
    u = np.maximum(np.float32(0.0), u)
    u = np.clip(u, _EPS, 1.0 - _EPS).astype(np.float32)
    return (-np.log(-np.log(u))).reshape(_B, _N)


_NOISE_CONST = _gumbel_noise()


_GV = 8
_NG = _NVREG // _GV


def _sc_topk_body(x_hbm, d_hbm, row_v, out_v, sum_v):
    wid = lax.axis_index("s") * 2 + lax.axis_index("c")
    idx15 = jnp.full((_LANES,), _LANES - 1, jnp.int32)

    for rr in range(_ROWS_PER_W):
        r = wid * _ROWS_PER_W + rr
        pltpu.sync_copy(x_hbm.at[r], row_v)

        def p1(g, _):
            base = g * (_GV * _LANES)
            gm = row_v[pl.ds(base, _LANES)]
            for j in range(1, _GV):
                gm = jnp.maximum(gm, row_v[pl.ds(base + j * _LANES, _LANES)])
            sum_v[pl.ds(g * _LANES, _LANES)] = gm
            return 0

        lax.fori_loop(0, _NG, p1, 0)

        def p2(g, carry):
            cand, cmin = carry
            v = sum_v[pl.ds(g * _LANES, _LANES)]

            def merge(args):
                cand, _ = args
                vs = plsc.sort_key_val(v, v)[0]
                m = jnp.maximum(cand, vs)
                cand2 = plsc.sort_key_val(m, m, descending=True)[0]
                return cand2, jnp.take(cand2, idx15)

            def skip(args):
                return args

            return lax.cond(jnp.any(v >= cmin), merge, skip, (cand, cmin))

        cand0 = jnp.full((_LANES,), _NEG, jnp.float32)
        _, c16 = lax.fori_loop(0, _NG, p2,
                               (cand0, jnp.full((_LANES,), _NEG, jnp.float32)))

        def p3(g, cand):
            gm = sum_v[pl.ds(g * _LANES, _LANES)]

            def hit(cand):
                base = g * (_GV * _LANES)
                for j in range(_GV):
                    v = row_v[pl.ds(base + j * _LANES, _LANES)]
                    vs = plsc.sort_key_val(v, v)[0]
                    m = jnp.maximum(cand, vs)
                    cand = plsc.sort_key_val(m, m, descending=True)[0]
                return cand

            def skip(cand):
                return cand

            return lax.cond(jnp.any(gm >= c16), hit, skip, cand)

        cande = lax.fori_loop(0, _NG, p3, cand0)
        thr = jnp.take(cande, idx15)

        def p4(g, _):
            base = g * (_GV * _LANES)
            for j in range(_GV):
                sl = pl.ds(base + j * _LANES, _LANES)
                out_v[sl] = jnp.where(row_v[sl] >= thr,
                                      jnp.float32(1.0), jnp.float32(0.0))
            return 0

        lax.fori_loop(0, _NG, p4, 0)
        pltpu.sync_copy(out_v, d_hbm.at[r])


def _sc_topk(x):
    mesh = plsc.VectorSubcoreMesh(core_axis_name="c", subcore_axis_name="s")
    fn = functools.partial(
        pl.kernel,
        mesh=mesh,
        out_type=jax.ShapeDtypeStruct((_B, _N), jnp.float32),
        scratch_types=[
            pltpu.VMEM((_N,), jnp.float32),
            pltpu.VMEM((_N,), jnp.float32),
            pltpu.VMEM((_NG * _LANES,), jnp.float32),
        ],
        compiler_params=pltpu.CompilerParams(needs_layout_passes=False),
    )(_sc_topk_body)
    return fn(x)


def _soft_sub_kernel(x_ref, z_ref, c_ref, w_ref):
    w0 = x_ref[...] + z_ref[...]
    w_ref[...] = w0
    c_ref[...] = jnp.zeros((_ROWS, _N), jnp.float32)
    mw0 = jnp.max(w0, axis=1, keepdims=True)

    def cbody(_, mw):
        mws = mw / _T
        w = w_ref[...]
        s = jnp.sum(jnp.exp(w / _T - mws), axis=1, keepdims=True)
        w2 = w_ref[...]
        oh = jnp.exp(w2 / _T - mws) / s
        c_ref[...] = c_ref[...] + oh
        wn = w2 + jnp.log(jnp.clip(1.0 - oh, _EPS, 1.0 - _EPS))
        w_ref[...] = wn
        return jnp.max(wn, axis=1, keepdims=True)

    jax.lax.fori_loop(0, _K, cbody, mw0)


def _tc_relax(x, z):
    grid = (_B // _ROWS,)
    return pl.pallas_call(
        _soft_sub_kernel,
        grid=grid,
        in_specs=[
            pl.BlockSpec((_ROWS, _N), lambda i: (i, 0)),
            pl.BlockSpec((_ROWS, _N), lambda i: (i, 0)),
        ],
        out_specs=pl.BlockSpec((_ROWS, _N), lambda i: (i, 0)),
        out_shape=jax.ShapeDtypeStruct((_B, _N), jnp.float32),
        scratch_shapes=[
            pltpu.VMEM((_ROWS, _N), jnp.float32),
        ],
    )(x, z)


def kernel(logits):
    x = logits.reshape(_B, _N)
    z = jnp.asarray(_NOISE_CONST)
    dsamples = _sc_topk(x)
    csamples = _tc_relax(x, z)
    return dsamples, csamples
